# baseline (speedup 1.0000x reference)
import functools

import jax
import jax.numpy as jnp
from jax import lax
from jax.experimental import pallas as pl
from jax.experimental.pallas import tpu as pltpu
from jax.experimental.pallas import tpu_sc as plsc

C = 128
L = 16
NB = 2


def _make_sc_conv(N, D, E_pad):
    info = plsc.get_sparse_core_info()
    NC, NS = info.num_cores, info.num_subcores
    NW = NC * NS
    n_chunks = E_pad // C
    cpt = n_chunks // NW
    assert n_chunks % NW == 0 and cpt % NB == 0 and N % L == 0

    n_full = N // C
    tail = N - n_full * C
    n_row_chunks = n_full + (1 if tail else 0)
    jj_iters = -(-n_row_chunks // NS)

    mesh = plsc.VectorSubcoreMesh(core_axis_name="c", subcore_axis_name="s")

    @functools.partial(
        pl.kernel,
        out_type=(
            jax.ShapeDtypeStruct((NC, N, D), jnp.float32),
            jax.ShapeDtypeStruct((NW * N,), jnp.float32),
        ),
        mesh=mesh,
        compiler_params=pltpu.CompilerParams(needs_layout_passes=False),
        scratch_types=[
            [pltpu.VMEM((C,), jnp.int32)] * NB,
            [pltpu.VMEM((C,), jnp.int32)] * NB,
            [pltpu.VMEM((C,), jnp.float32)] * NB,
            [pltpu.VMEM((C, D), jnp.float32)] * NB,
            pltpu.VMEM((N,), jnp.float32),
            pltpu.VMEM_SHARED((N, D), jnp.float32),
            pltpu.SemaphoreType.DMA,
            [pltpu.SemaphoreType.DMA] * NB,
        ],
    )
    def conv(x_hbm, src_hbm, dst_hbm, ea_hbm, y_hbm, s_hbm,
             src_b, dst_b, ea_b, rows, s_acc, y_sh, gsem, ssem):
        cid = lax.axis_index("c")
        sid = lax.axis_index("s")
        wid = sid * NC + cid

        zero16 = jnp.zeros((L,), jnp.float32)
        bounce = rows[0]

        def zrow(r, _):
            for k in range(D // L):
                bounce[r, pl.ds(k * L, L)] = zero16
            return 0
        lax.fori_loop(0, C, zrow, 0)

        for jj in range(jj_iters):
            ch = sid + jj * NS
            rr = pl.multiple_of(ch * C, C)

            @pl.when(ch < n_full)
            def _():
                pltpu.sync_copy(bounce, y_sh.at[pl.ds(rr, C)])
            if tail:
                @pl.when(ch == n_full)
                def _():
                    pltpu.sync_copy(bounce.at[pl.ds(0, tail)],
                                    y_sh.at[pl.ds(n_full * C, tail)])

        def zs(i, _):
            s_acc[pl.ds(i * L, L)] = zero16
            return 0
        lax.fori_loop(0, N // L, zs, 0)

        plsc.subcore_barrier()

        def s_wait(b):
            pltpu.make_async_copy(rows[b], y_sh.at[pl.ds(0, C)], ssem[b]).wait()

        def outer(o, _):
            for b in range(NB):
                i = o * NB + b
                ch = wid + i * NW
                eb = pl.multiple_of(ch * C, C)

                pltpu.sync_copy(src_hbm.at[pl.ds(eb, C)], src_b[b])
                pltpu.sync_copy(dst_hbm.at[pl.ds(eb, C)], dst_b[b])
                pltpu.sync_copy(ea_hbm.at[pl.ds(eb, C)], ea_b[b])
                pltpu.async_copy(x_hbm.at[src_b[b]], rows[b], gsem).wait()

                sb, eab, rb = src_b[b], ea_b[b], rows[b]

                def scale16(j, _):
                    src16 = sb[pl.ds(j * L, L)]
                    ea16 = eab[pl.ds(j * L, L)]
                    plsc.addupdate_scatter(s_acc, [src16], ea16)
                    for t in range(L):
                        e = j * L + t
                        bc = plsc.load_gather(eab, [jnp.full((L,), e, jnp.int32)])
                        for k in range(D // L):
                            rb[e, pl.ds(k * L, L)] = rb[e, pl.ds(k * L, L)] * bc
                    return 0
                lax.fori_loop(0, C // L, scale16, 0)

                pltpu.sync_copy(rb, y_sh.at[dst_b[b]], add=True)
            return 0
        lax.fori_loop(0, cpt // NB, outer, 0)

        plsc.subcore_barrier()

        for jj in range(jj_iters):
            ch = sid + jj * NS
            rr = pl.multiple_of(ch * C, C)

            @pl.when(ch < n_full)
            def _():
                pltpu.sync_copy(y_sh.at[pl.ds(rr, C)], bounce)
                pltpu.sync_copy(bounce, y_hbm.at[cid, pl.ds(rr, C)])
            if tail:
                @pl.when(ch == n_full)
                def _():
                    pltpu.sync_copy(y_sh.at[pl.ds(n_full * C, tail)],
                                    bounce.at[pl.ds(0, tail)])
                    pltpu.sync_copy(bounce.at[pl.ds(0, tail)],
                                    y_hbm.at[cid, pl.ds(n_full * C, tail)])
        pltpu.sync_copy(s_acc, s_hbm.at[pl.ds(pl.multiple_of(wid * N, 8), N)])

    return conv


def _tc_dense(y_part, s_part, x, ea_self, W1, b1, W2, b2):
    N, D = x.shape
    H = W1.shape[0]

    def body(yp, sp, xb, eas, W1r, b1r, W2r, b2r, out):
        y = yp[0] + yp[1] + eas[...] * xb[...]
        h1 = lax.dot_general(y, W1r[...], (((1,), (1,)), ((), ())),
                             preferred_element_type=jnp.float32)
        h1 = jnp.maximum(h1 + b1r[...], 0.0)
        stot = jnp.sum(sp[...], axis=0)[:, None] + eas[...]
        v = jnp.sum(stot * h1, axis=0, keepdims=True) * (1.0 / N)
        out[...] = lax.dot_general(v, W2r[...], (((1,), (1,)), ((), ())),
                                   preferred_element_type=jnp.float32) + b2r[...]

    return pl.pallas_call(
        body,
        out_shape=jax.ShapeDtypeStruct((1, H), jnp.float32),
    )(y_part, s_part, x, ea_self, W1, b1.reshape(1, H), W2, b2.reshape(1, H))


def kernel(x, edge_index, edge_attr, W1, b1, W2, b2):
    N, D = x.shape
    E = edge_index.shape[1]

    info = plsc.get_sparse_core_info()
    NW = info.num_cores * info.num_subcores

    unit = C * NW * NB
    E_pad = -(-E // unit) * unit
    pad = E_pad - E
    src = jnp.concatenate([edge_index[0], jnp.zeros((pad,), jnp.int32)])
    dst = jnp.concatenate([edge_index[1], jnp.zeros((pad,), jnp.int32)])
    ea_e = jnp.concatenate([edge_attr[:E], jnp.zeros((pad,), jnp.float32)])
    ea_self = edge_attr[E:].reshape(N, 1)

    conv = _make_sc_conv(N, D, E_pad)
    y_part, s_flat = conv(x, src, dst, ea_e)
    s_part = s_flat.reshape(NW, N)
    out = _tc_dense(y_part, s_part, x, ea_self, W1, b1, W2, b2)
    return out.reshape(D)


# speedup vs baseline: 1.6470x; 1.6470x over previous
import functools

import jax
import jax.numpy as jnp
from jax import lax
from jax.experimental import pallas as pl
from jax.experimental.pallas import tpu as pltpu
from jax.experimental.pallas import tpu_sc as plsc

C = 128
L = 16


def _make_sc_conv(N, D, E):
    info = plsc.get_sparse_core_info()
    NC, NS = info.num_cores, info.num_subcores
    NW = NC * NS
    n_chunks = E // C
    assert E % C == 0 and N % L == 0

    n_full = N // C
    tail = N - n_full * C
    n_row_chunks = n_full + (1 if tail else 0)
    jj_iters = -(-n_row_chunks // NS)

    mesh = plsc.VectorSubcoreMesh(core_axis_name="c", subcore_axis_name="s")

    @functools.partial(
        pl.kernel,
        out_type=(
            jax.ShapeDtypeStruct((NC, N, D), jnp.float32),
            jax.ShapeDtypeStruct((NW * N,), jnp.float32),
        ),
        mesh=mesh,
        compiler_params=pltpu.CompilerParams(needs_layout_passes=False),
        scratch_types=[
            pltpu.VMEM((C,), jnp.int32),
            pltpu.VMEM((C,), jnp.int32),
            pltpu.VMEM((C,), jnp.float32),
            pltpu.VMEM((C, D), jnp.float32),
            pltpu.VMEM((N,), jnp.float32),
            pltpu.VMEM_SHARED((N, D), jnp.float32),
            pltpu.SemaphoreType.DMA,
        ],
    )
    def conv(x_hbm, src_hbm, dst_hbm, ea_hbm, y_hbm, s_hbm,
             src_v, dst_v, ea_v, rows_v, s_acc, y_sh, sem):
        cid = lax.axis_index("c")
        sid = lax.axis_index("s")
        wid = sid * NC + cid

        zero16 = jnp.zeros((L,), jnp.float32)

        def zrow(r, _):
            for k in range(D // L):
                rows_v[r, pl.ds(k * L, L)] = zero16
            return 0
        lax.fori_loop(0, C, zrow, 0)

        for jj in range(jj_iters):
            ch = sid + jj * NS
            r0 = pl.multiple_of(ch * C, C)

            @pl.when(ch < n_full)
            def _():
                pltpu.sync_copy(rows_v, y_sh.at[pl.ds(r0, C)])
            if tail:
                @pl.when(ch == n_full)
                def _():
                    pltpu.sync_copy(rows_v.at[pl.ds(0, tail)],
                                    y_sh.at[pl.ds(n_full * C, tail)])

        def zs(i, _):
            s_acc[pl.ds(i * L, L)] = zero16
            return 0
        lax.fori_loop(0, N // L, zs, 0)

        plsc.subcore_barrier()

        base = n_chunks // NW
        extra = n_chunks % NW
        cnt = base + jnp.where(wid < extra, 1, 0)

        def chunk_body(i, _):
            ch = wid + i * NW
            eb = pl.multiple_of(ch * C, C)
            pltpu.sync_copy(src_hbm.at[pl.ds(eb, C)], src_v)
            pltpu.sync_copy(dst_hbm.at[pl.ds(eb, C)], dst_v)
            pltpu.sync_copy(ea_hbm.at[pl.ds(eb, C)], ea_v)
            pltpu.async_copy(x_hbm.at[src_v], rows_v, sem).wait()

            def scale16(j, _):
                src16 = src_v[pl.ds(j * L, L)]
                ea16 = ea_v[pl.ds(j * L, L)]
                plsc.addupdate_scatter(s_acc, [src16], ea16)
                for t in range(L):
                    e = j * L + t
                    b = plsc.load_gather(ea_v, [jnp.full((L,), e, jnp.int32)])
                    for k in range(D // L):
                        rows_v[e, pl.ds(k * L, L)] = rows_v[e, pl.ds(k * L, L)] * b
                return 0
            lax.fori_loop(0, C // L, scale16, 0)

            pltpu.sync_copy(rows_v, y_sh.at[dst_v], add=True)
            return 0
        lax.fori_loop(0, cnt, chunk_body, 0)

        plsc.subcore_barrier()

        for jj in range(jj_iters):
            ch = sid + jj * NS
            r0 = pl.multiple_of(ch * C, C)

            @pl.when(ch < n_full)
            def _():
                pltpu.sync_copy(y_sh.at[pl.ds(r0, C)], rows_v)
                pltpu.sync_copy(rows_v, y_hbm.at[cid, pl.ds(r0, C)])
            if tail:
                @pl.when(ch == n_full)
                def _():
                    pltpu.sync_copy(y_sh.at[pl.ds(n_full * C, tail)],
                                    rows_v.at[pl.ds(0, tail)])
                    pltpu.sync_copy(rows_v.at[pl.ds(0, tail)],
                                    y_hbm.at[cid, pl.ds(n_full * C, tail)])
        pltpu.sync_copy(s_acc, s_hbm.at[pl.ds(pl.multiple_of(wid * N, 8), N)])

    return conv


def _tc_dense(y_part, s_part, x, ea_self, W1, b1, W2, b2):
    N, D = x.shape
    H = W1.shape[0]

    def body(yp, sp, xb, eas, W1r, b1r, W2r, b2r, out):
        y = yp[0] + yp[1] + eas[...] * xb[...]
        h1 = lax.dot_general(y, W1r[...], (((1,), (1,)), ((), ())),
                             preferred_element_type=jnp.float32)
        h1 = jnp.maximum(h1 + b1r[...], 0.0)
        stot = jnp.sum(sp[...], axis=0)[:, None] + eas[...]
        v = jnp.sum(stot * h1, axis=0, keepdims=True) * (1.0 / N)
        out[...] = lax.dot_general(v, W2r[...], (((1,), (1,)), ((), ())),
                                   preferred_element_type=jnp.float32) + b2r[...]

    return pl.pallas_call(
        body,
        out_shape=jax.ShapeDtypeStruct((1, H), jnp.float32),
    )(y_part, s_part, x, ea_self, W1, b1.reshape(1, H), W2, b2.reshape(1, H))


def kernel(x, edge_index, edge_attr, W1, b1, W2, b2):
    N, D = x.shape
    E = edge_index.shape[1]
    src = edge_index[0]
    dst = edge_index[1]
    ea_e = edge_attr[:E]
    ea_self = edge_attr[E:].reshape(N, 1)

    info = plsc.get_sparse_core_info()
    NW = info.num_cores * info.num_subcores

    conv = _make_sc_conv(N, D, E)
    y_part, s_flat = conv(x, src, dst, ea_e)
    s_part = s_flat.reshape(NW, N)
    out = _tc_dense(y_part, s_part, x, ea_self, W1, b1, W2, b2)
    return out.reshape(D)
